# baseline (speedup 1.0000x reference)
import functools

import jax
import jax.numpy as jnp
from jax import lax
from jax.experimental import pallas as pl
from jax.experimental.pallas import tpu as pltpu
from jax.experimental.pallas import tpu_sc as plsc

N = 10000
NPAD = 10240
E = 320000
D1 = 128
D2 = 64
NC = 2
NS = 16
NT = NC * NS
K = 100
NB = 100
NBUF = 8
DEGW = 8
DEG_DEPTH = 8
RPT = NPAD // NS

_sc_mesh = plsc.VectorSubcoreMesh(core_axis_name="c", subcore_axis_name="s")


def _make_agg(D, with_deg=False):
    def body(*args):
        if with_deg:
            (y, row2d, col2d, zeros, ones, zeros8, out, degout,
             ridx, cidx, acc, bufs, onesv, dacc,
             gsems, ssems, dsem) = args
        else:
            (y, row2d, col2d, zeros, out, ridx, cidx, acc, bufs,
             gsems, ssems) = args
        c = lax.axis_index("c")
        s = lax.axis_index("s")
        tid = c * NS + s

        pltpu.sync_copy(row2d.at[tid], ridx)
        pltpu.sync_copy(col2d.at[tid], cidx)
        base = s * RPT
        pltpu.sync_copy(zeros, acc.at[pl.ds(base, RPT)])
        if with_deg:
            pltpu.sync_copy(ones, onesv)
            pltpu.sync_copy(zeros8, dacc.at[pl.ds(base, RPT)])
        plsc.subcore_barrier()

        def pstep(b, carry):
            pltpu.make_async_copy(
                y.at[ridx.at[b]], bufs.at[b], gsems.at[b]).start()
            return carry

        lax.fori_loop(0, NBUF, pstep, 0)

        def step(i, carry):
            @pl.when(i < NB)
            def _work():
                b = lax.rem(i, NBUF)
                pltpu.make_async_copy(y.at[ridx.at[i]], bufs.at[b],
                                      gsems.at[b]).wait()
                pltpu.make_async_copy(bufs.at[b], acc.at[cidx.at[i]],
                                      ssems.at[b]).start(add=True)
                if with_deg:
                    pltpu.make_async_copy(onesv, dacc.at[cidx.at[i]],
                                          dsem).start(add=True)

                    @pl.when(i >= DEG_DEPTH)
                    def _ddrain():
                        pltpu.make_async_copy(
                            onesv, dacc.at[cidx.at[i - DEG_DEPTH]],
                            dsem).wait()

            @pl.when(i >= 1)
            def _retire():
                pi = i - 1
                pb = lax.rem(pi, NBUF)
                pltpu.make_async_copy(bufs.at[pb], acc.at[cidx.at[pi]],
                                      ssems.at[pb]).wait()

                @pl.when(pi + NBUF < NB)
                def _refill():
                    pltpu.make_async_copy(y.at[ridx.at[pi + NBUF]],
                                          bufs.at[pb], gsems.at[pb]).start()

            return carry

        lax.fori_loop(0, NB + 1, step, 0)
        if with_deg:
            def dstep(j, carry):
                pltpu.make_async_copy(onesv, dacc.at[cidx.at[j]],
                                      dsem).wait()
                return carry

            lax.fori_loop(0, DEG_DEPTH, dstep, 0)

        plsc.subcore_barrier()
        pltpu.sync_copy(acc.at[pl.ds(base, RPT)],
                        out.at[c].at[pl.ds(base, RPT)])
        if with_deg:
            pltpu.sync_copy(dacc.at[pl.ds(base, RPT)],
                            degout.at[c].at[pl.ds(base, RPT)])

    out_type = [jax.ShapeDtypeStruct((NC, NPAD, D), jnp.float32)]
    scratch = [
        pltpu.VMEM((NB, K), jnp.int32),
        pltpu.VMEM((NB, K), jnp.int32),
        pltpu.VMEM_SHARED((NPAD, D), jnp.float32),
        pltpu.VMEM((NBUF, K, D), jnp.float32),
    ]
    if with_deg:
        out_type.append(jax.ShapeDtypeStruct((NC, NPAD, DEGW), jnp.float32))
        scratch += [
            pltpu.VMEM((K, DEGW), jnp.float32),
            pltpu.VMEM_SHARED((NPAD, DEGW), jnp.float32),
        ]
    scratch += [
        pltpu.SemaphoreType.DMA((NBUF,)),
        pltpu.SemaphoreType.DMA((NBUF,)),
    ]
    if with_deg:
        scratch.append(pltpu.SemaphoreType.DMA)
    return pl.kernel(
        body,
        out_type=out_type if with_deg else out_type[0],
        scratch_types=scratch,
        mesh=_sc_mesh,
        compiler_params=pltpu.CompilerParams(use_tc_tiling_on_sc=False),
    )


_agg64 = _make_agg(D2)
_agg64d = _make_agg(D2, with_deg=True)


def _layer_body(pa_ref, pb_ref, d_ref, b_ref, w1a_ref, w1b_ref, w2_ref,
                o_ref):
    d = jnp.maximum(d_ref[0, :, 0:1] + d_ref[1, :, 0:1], 1.0)
    ha = (pa_ref[0] + pa_ref[1]) / d
    hb = (pb_ref[0] + pb_ref[1]) / d
    h = jnp.maximum(
        jnp.dot(ha, w1a_ref[...], preferred_element_type=jnp.float32)
        + jnp.dot(hb, w1b_ref[...], preferred_element_type=jnp.float32)
        + b_ref[...], 0.0)
    o_ref[...] = jnp.dot(h, w2_ref[...], preferred_element_type=jnp.float32)


def _out_body(p_ref, d_ref, b2_ref, w_ref, b3_ref, o_ref):
    p = p_ref[0] + p_ref[1]
    d = jnp.maximum(d_ref[0, :, 0:1] + d_ref[1, :, 0:1], 1.0)
    h = jnp.maximum(p / d + b2_ref[...], 0.0)
    o_ref[...] = (jnp.dot(h, w_ref[...], preferred_element_type=jnp.float32)
                  + b3_ref[...])


def _layer(pa, pb, degp, b1, w1a, w1b, w2):
    blk = NPAD // 10
    return pl.pallas_call(
        _layer_body,
        grid=(10,),
        in_specs=[
            pl.BlockSpec((NC, blk, D2), lambda i: (0, i, 0)),
            pl.BlockSpec((NC, blk, D2), lambda i: (0, i, 0)),
            pl.BlockSpec((NC, blk, DEGW), lambda i: (0, i, 0)),
            pl.BlockSpec((1, D1), lambda i: (0, 0)),
            pl.BlockSpec((D2, D1), lambda i: (0, 0)),
            pl.BlockSpec((D2, D1), lambda i: (0, 0)),
            pl.BlockSpec((D1, D2), lambda i: (0, 0)),
        ],
        out_specs=pl.BlockSpec((blk, D2), lambda i: (i, 0)),
        out_shape=jax.ShapeDtypeStruct((NPAD, D2), jnp.float32),
    )(pa, pb, degp, b1, w1a, w1b, w2)


def _out_layer(part, degp, b2, w, b3):
    din = part.shape[2]
    dout = w.shape[1]
    blk = N // 10
    return pl.pallas_call(
        _out_body,
        grid=(10,),
        in_specs=[
            pl.BlockSpec((NC, blk, din), lambda i: (0, i, 0)),
            pl.BlockSpec((NC, blk, DEGW), lambda i: (0, i, 0)),
            pl.BlockSpec((1, din), lambda i: (0, 0)),
            pl.BlockSpec((din, dout), lambda i: (0, 0)),
            pl.BlockSpec((1, dout), lambda i: (0, 0)),
        ],
        out_specs=pl.BlockSpec((blk, dout), lambda i: (i, 0)),
        out_shape=jax.ShapeDtypeStruct((N, dout), jnp.float32),
    )(part, degp, b2, w, b3)


def kernel(x, edge_index, n_nodes, W1, b1, W2, b2, W3, b3):
    row2d = edge_index[0].reshape(NT, NB, K)
    col2d = edge_index[1].reshape(NT, NB, K)
    zeros2 = jnp.zeros((RPT, D2), jnp.float32)
    zeros8 = jnp.zeros((RPT, DEGW), jnp.float32)
    ones8 = jnp.ones((K, DEGW), jnp.float32)
    xa = x[:, :D2]
    xb = x[:, D2:]

    pa, degp = _agg64d(xa, row2d, col2d, zeros2, ones8, zeros8)
    pb = _agg64(xb, row2d, col2d, zeros2)
    y2 = _layer(pa, pb, degp, b1.reshape(1, D1),
                W1[:D2, :], W1[D2:, :], W2)
    part2 = _agg64(y2, row2d, col2d, zeros2)
    out = _out_layer(part2, degp, b2.reshape(1, D2), W3, b3.reshape(1, D2))
    return out


# speedup vs baseline: 1.0045x; 1.0045x over previous
import functools

import jax
import jax.numpy as jnp
from jax import lax
from jax.experimental import pallas as pl
from jax.experimental.pallas import tpu as pltpu
from jax.experimental.pallas import tpu_sc as plsc

N = 10000
NPAD = 10240
E = 320000
D1 = 128
D2 = 64
NC = 2
NS = 16
NT = NC * NS
K = 100
NB = 100
NBUF = 8
DEGW = 8
DEG_DEPTH = 8
RLAG = 4
RPT = NPAD // NS

_sc_mesh = plsc.VectorSubcoreMesh(core_axis_name="c", subcore_axis_name="s")


def _make_agg(D, with_deg=False):
    def body(*args):
        if with_deg:
            (y, row2d, col2d, zeros, ones, zeros8, out, degout,
             ridx, cidx, acc, bufs, onesv, dacc,
             gsems, ssems, dsem) = args
        else:
            (y, row2d, col2d, zeros, out, ridx, cidx, acc, bufs,
             gsems, ssems) = args
        c = lax.axis_index("c")
        s = lax.axis_index("s")
        tid = c * NS + s

        pltpu.sync_copy(row2d.at[tid], ridx)
        pltpu.sync_copy(col2d.at[tid], cidx)
        base = s * RPT
        pltpu.sync_copy(zeros, acc.at[pl.ds(base, RPT)])
        if with_deg:
            pltpu.sync_copy(ones, onesv)
            pltpu.sync_copy(zeros8, dacc.at[pl.ds(base, RPT)])
        plsc.subcore_barrier()

        def pstep(b, carry):
            pltpu.make_async_copy(
                y.at[ridx.at[b]], bufs.at[b], gsems.at[b]).start()
            return carry

        lax.fori_loop(0, NBUF, pstep, 0)

        def step(i, carry):
            @pl.when(i < NB)
            def _work():
                b = lax.rem(i, NBUF)
                pltpu.make_async_copy(y.at[ridx.at[i]], bufs.at[b],
                                      gsems.at[b]).wait()
                pltpu.make_async_copy(bufs.at[b], acc.at[cidx.at[i]],
                                      ssems.at[b]).start(add=True)
                if with_deg:
                    pltpu.make_async_copy(onesv, dacc.at[cidx.at[i]],
                                          dsem).start(add=True)

                    @pl.when(i >= DEG_DEPTH)
                    def _ddrain():
                        pltpu.make_async_copy(
                            onesv, dacc.at[cidx.at[i - DEG_DEPTH]],
                            dsem).wait()

            @pl.when(i >= RLAG)
            def _retire():
                pi = i - RLAG
                pb = lax.rem(pi, NBUF)
                pltpu.make_async_copy(bufs.at[pb], acc.at[cidx.at[pi]],
                                      ssems.at[pb]).wait()

                @pl.when(pi + NBUF < NB)
                def _refill():
                    pltpu.make_async_copy(y.at[ridx.at[pi + NBUF]],
                                          bufs.at[pb], gsems.at[pb]).start()

            return carry

        lax.fori_loop(0, NB + RLAG, step, 0)
        if with_deg:
            def dstep(j, carry):
                pltpu.make_async_copy(onesv, dacc.at[cidx.at[j]],
                                      dsem).wait()
                return carry

            lax.fori_loop(0, DEG_DEPTH, dstep, 0)

        plsc.subcore_barrier()
        pltpu.sync_copy(acc.at[pl.ds(base, RPT)],
                        out.at[c].at[pl.ds(base, RPT)])
        if with_deg:
            pltpu.sync_copy(dacc.at[pl.ds(base, RPT)],
                            degout.at[c].at[pl.ds(base, RPT)])

    out_type = [jax.ShapeDtypeStruct((NC, NPAD, D), jnp.float32)]
    scratch = [
        pltpu.VMEM((NB, K), jnp.int32),
        pltpu.VMEM((NB, K), jnp.int32),
        pltpu.VMEM_SHARED((NPAD, D), jnp.float32),
        pltpu.VMEM((NBUF, K, D), jnp.float32),
    ]
    if with_deg:
        out_type.append(jax.ShapeDtypeStruct((NC, NPAD, DEGW), jnp.float32))
        scratch += [
            pltpu.VMEM((K, DEGW), jnp.float32),
            pltpu.VMEM_SHARED((NPAD, DEGW), jnp.float32),
        ]
    scratch += [
        pltpu.SemaphoreType.DMA((NBUF,)),
        pltpu.SemaphoreType.DMA((NBUF,)),
    ]
    if with_deg:
        scratch.append(pltpu.SemaphoreType.DMA)
    return pl.kernel(
        body,
        out_type=out_type if with_deg else out_type[0],
        scratch_types=scratch,
        mesh=_sc_mesh,
        compiler_params=pltpu.CompilerParams(use_tc_tiling_on_sc=False),
    )


def _deg_body(col2d, ones, zeros8, degout, cidx, onesv, dacc, dsem):
    c = lax.axis_index("c")
    s = lax.axis_index("s")
    tid = c * NS + s
    pltpu.sync_copy(col2d.at[tid], cidx)
    pltpu.sync_copy(ones, onesv)
    base = s * RPT
    pltpu.sync_copy(zeros8, dacc.at[pl.ds(base, RPT)])
    plsc.subcore_barrier()

    def step(i, carry):
        pltpu.make_async_copy(onesv, dacc.at[cidx.at[i]],
                              dsem).start(add=True)

        @pl.when(i >= DEG_DEPTH)
        def _drain():
            pltpu.make_async_copy(onesv, dacc.at[cidx.at[i - DEG_DEPTH]],
                                  dsem).wait()

        return carry

    lax.fori_loop(0, NB, step, 0)

    def dstep(j, carry):
        pltpu.make_async_copy(onesv, dacc.at[cidx.at[j]], dsem).wait()
        return carry

    lax.fori_loop(0, DEG_DEPTH, dstep, 0)

    plsc.subcore_barrier()
    pltpu.sync_copy(dacc.at[pl.ds(base, RPT)],
                    degout.at[c].at[pl.ds(base, RPT)])


_deg = pl.kernel(
    _deg_body,
    out_type=jax.ShapeDtypeStruct((NC, NPAD, DEGW), jnp.float32),
    scratch_types=[
        pltpu.VMEM((NB, K), jnp.int32),
        pltpu.VMEM((K, DEGW), jnp.float32),
        pltpu.VMEM_SHARED((NPAD, DEGW), jnp.float32),
        pltpu.SemaphoreType.DMA,
    ],
    mesh=_sc_mesh,
    compiler_params=pltpu.CompilerParams(use_tc_tiling_on_sc=False),
)


_agg64 = _make_agg(D2)


def _layer_body(pa_ref, pb_ref, d_ref, b_ref, w1a_ref, w1b_ref, w2_ref,
                o_ref):
    d = jnp.maximum(d_ref[0, :, 0:1] + d_ref[1, :, 0:1], 1.0)
    ha = (pa_ref[0] + pa_ref[1]) / d
    hb = (pb_ref[0] + pb_ref[1]) / d
    h = jnp.maximum(
        jnp.dot(ha, w1a_ref[...], preferred_element_type=jnp.float32)
        + jnp.dot(hb, w1b_ref[...], preferred_element_type=jnp.float32)
        + b_ref[...], 0.0)
    o_ref[...] = jnp.dot(h, w2_ref[...], preferred_element_type=jnp.float32)


def _out_body(p_ref, d_ref, b2_ref, w_ref, b3_ref, o_ref):
    p = p_ref[0] + p_ref[1]
    d = jnp.maximum(d_ref[0, :, 0:1] + d_ref[1, :, 0:1], 1.0)
    h = jnp.maximum(p / d + b2_ref[...], 0.0)
    o_ref[...] = (jnp.dot(h, w_ref[...], preferred_element_type=jnp.float32)
                  + b3_ref[...])


def _layer(pa, pb, degp, b1, w1a, w1b, w2):
    blk = NPAD // 10
    return pl.pallas_call(
        _layer_body,
        grid=(10,),
        in_specs=[
            pl.BlockSpec((NC, blk, D2), lambda i: (0, i, 0)),
            pl.BlockSpec((NC, blk, D2), lambda i: (0, i, 0)),
            pl.BlockSpec((NC, blk, DEGW), lambda i: (0, i, 0)),
            pl.BlockSpec((1, D1), lambda i: (0, 0)),
            pl.BlockSpec((D2, D1), lambda i: (0, 0)),
            pl.BlockSpec((D2, D1), lambda i: (0, 0)),
            pl.BlockSpec((D1, D2), lambda i: (0, 0)),
        ],
        out_specs=pl.BlockSpec((blk, D2), lambda i: (i, 0)),
        out_shape=jax.ShapeDtypeStruct((NPAD, D2), jnp.float32),
    )(pa, pb, degp, b1, w1a, w1b, w2)


def _out_layer(part, degp, b2, w, b3):
    din = part.shape[2]
    dout = w.shape[1]
    blk = N // 10
    return pl.pallas_call(
        _out_body,
        grid=(10,),
        in_specs=[
            pl.BlockSpec((NC, blk, din), lambda i: (0, i, 0)),
            pl.BlockSpec((NC, blk, DEGW), lambda i: (0, i, 0)),
            pl.BlockSpec((1, din), lambda i: (0, 0)),
            pl.BlockSpec((din, dout), lambda i: (0, 0)),
            pl.BlockSpec((1, dout), lambda i: (0, 0)),
        ],
        out_specs=pl.BlockSpec((blk, dout), lambda i: (i, 0)),
        out_shape=jax.ShapeDtypeStruct((N, dout), jnp.float32),
    )(part, degp, b2, w, b3)


def kernel(x, edge_index, n_nodes, W1, b1, W2, b2, W3, b3):
    row2d = edge_index[0].reshape(NT, NB, K)
    col2d = edge_index[1].reshape(NT, NB, K)
    zeros2 = jnp.zeros((RPT, D2), jnp.float32)
    zeros8 = jnp.zeros((RPT, DEGW), jnp.float32)
    ones8 = jnp.ones((K, DEGW), jnp.float32)
    xa = x[:, :D2]
    xb = x[:, D2:]

    degp = _deg(col2d, ones8, zeros8)
    pa = _agg64(xa, row2d, col2d, zeros2)
    pb = _agg64(xb, row2d, col2d, zeros2)
    y2 = _layer(pa, pb, degp, b1.reshape(1, D1),
                W1[:D2, :], W1[D2:, :], W2)
    part2 = _agg64(y2, row2d, col2d, zeros2)
    out = _out_layer(part2, degp, b2.reshape(1, D2), W3, b3.reshape(1, D2))
    return out
